# baseline (speedup 1.0000x reference)
import functools

import jax
import jax.numpy as jnp
from jax import lax
from jax.experimental import pallas as pl
from jax.experimental.pallas import tpu as pltpu
from jax.experimental.pallas import tpu_sc as plsc

B = 2
N = 10000
E = 160000
D = 128
NODE_DIM = 32
EPS = 1e-6

NC = 2
NS = 16
CH = 128
K = 80
TWO_K = 2 * K
KB = 40
E_PAD = NS * K * CH
N_PAD = 10112
ROWS_PER_SUB = N_PAD // NS
TRASH = N
R = B * N_PAD
CNT_W = 128


def _sc_mesh():
    return plsc.VectorSubcoreMesh(core_axis_name="c", subcore_axis_name="s")


def _sc_aggregate(t_flat, gat_idx, sct_idx, zeros_acc):
    @functools.partial(
        pl.kernel,
        mesh=_sc_mesh(),
        out_type=jax.ShapeDtypeStruct((B, N_PAD, D), jnp.float32),
        scratch_types=[
            pltpu.VMEM((KB, CH), jnp.int32),
            pltpu.VMEM((KB, CH), jnp.int32),
            pltpu.VMEM((CH, D), jnp.float32),
            pltpu.VMEM((CH, D), jnp.float32),
            pltpu.VMEM_SHARED((N_PAD, D), jnp.float32),
            pltpu.SemaphoreType.DMA,
        ],
    )
    def k(t_hbm, g_hbm, s_hbm, z_hbm, out_hbm, gi_v, si_v, buf0, buf1,
          acc_sh, gsem):
        b = lax.axis_index("c")
        s = lax.axis_index("s")
        row0 = s * ROWS_PER_SUB
        pltpu.sync_copy(z_hbm.at[pl.ds(row0, ROWS_PER_SUB)],
                        acc_sh.at[pl.ds(row0, ROWS_PER_SUB)])
        plsc.subcore_barrier()
        bufs = (buf0, buf1)
        for q in range(TWO_K // KB):
            pltpu.sync_copy(g_hbm.at[b].at[s].at[pl.ds(q * KB, KB)], gi_v)
            pltpu.sync_copy(s_hbm.at[b].at[s].at[pl.ds(q * KB, KB)], si_v)
            pltpu.async_copy(t_hbm.at[gi_v.at[0]], bufs[0], gsem)

            @pl.loop(0, KB, step=2)
            def _(jj):
                for i in range(2):
                    j = jj + i
                    buf, nbuf = bufs[i], bufs[1 - i]
                    pltpu.make_async_copy(
                        t_hbm.at[gi_v.at[j]], buf, gsem).wait()

                    @pl.when(j + 1 < KB)
                    def _():
                        pltpu.async_copy(t_hbm.at[gi_v.at[j + 1]], nbuf, gsem)

                    pltpu.sync_copy(buf, acc_sh.at[si_v.at[j]], add=True)

        plsc.subcore_barrier()
        pltpu.sync_copy(acc_sh.at[pl.ds(row0, ROWS_PER_SUB)],
                        out_hbm.at[b].at[pl.ds(row0, ROWS_PER_SUB)])

    return k(t_flat, gat_idx, sct_idx, zeros_acc)


def _sc_degree(cnt_idx, zeros_cnt, ones_rows):
    @functools.partial(
        pl.kernel,
        mesh=_sc_mesh(),
        out_type=jax.ShapeDtypeStruct((2, B, N_PAD, CNT_W), jnp.float32),
        scratch_types=[
            pltpu.VMEM((TWO_K, CH), jnp.int32),
            pltpu.VMEM((CH, CNT_W), jnp.float32),
            pltpu.VMEM_SHARED((N_PAD, CNT_W), jnp.float32),
            pltpu.SemaphoreType.DMA,
        ],
    )
    def k(i_hbm, z_hbm, o_hbm, out_hbm, idx_v, ones_v, cnt_sh, ssem):
        b = lax.axis_index("c")
        s = lax.axis_index("s")
        row0 = s * ROWS_PER_SUB
        pltpu.sync_copy(o_hbm, ones_v)
        for e in range(2):
            pltpu.sync_copy(i_hbm.at[e].at[b].at[s], idx_v)
            pltpu.sync_copy(z_hbm.at[pl.ds(row0, ROWS_PER_SUB)],
                            cnt_sh.at[pl.ds(row0, ROWS_PER_SUB)])
            plsc.subcore_barrier()

            @pl.loop(0, TWO_K)
            def _(j):
                pltpu.async_copy(ones_v, cnt_sh.at[idx_v.at[j]], ssem,
                                 add=True)

            @pl.loop(0, TWO_K)
            def _(j):
                pltpu.make_async_copy(
                    ones_v, cnt_sh.at[idx_v.at[j]], ssem).wait()

            plsc.subcore_barrier()
            pltpu.sync_copy(cnt_sh.at[pl.ds(row0, ROWS_PER_SUB)],
                            out_hbm.at[e].at[b].at[pl.ds(row0, ROWS_PER_SUB)])

    return k(cnt_idx, zeros_cnt, ones_rows)


TR = 1264


def _tc_init(nf, w_node, b_node, w0, b0):
    def body(nf_ref, wn_ref, bn_ref, w0_ref, b0_ref, t_ref):
        h = jnp.dot(nf_ref[...], wn_ref[...],
                    preferred_element_type=jnp.float32) + bn_ref[...]
        t_ref[...] = jnp.tanh(
            jnp.dot(h, w0_ref[...], preferred_element_type=jnp.float32)
            + b0_ref[...])

    return pl.pallas_call(
        body,
        grid=(R // TR,),
        in_specs=[
            pl.BlockSpec((TR, NODE_DIM), lambda i: (i, 0)),
            pl.BlockSpec((NODE_DIM, D), lambda i: (0, 0)),
            pl.BlockSpec((1, D), lambda i: (0, 0)),
            pl.BlockSpec((D, D), lambda i: (0, 0)),
            pl.BlockSpec((1, D), lambda i: (0, 0)),
        ],
        out_specs=pl.BlockSpec((TR, D), lambda i: (i, 0)),
        out_shape=jax.ShapeDtypeStruct((R, D), jnp.float32),
    )(nf, w_node, b_node, w0, b0)


def _tc_layer(acc, deg, w, b):
    def body(acc_ref, deg_ref, w_ref, b_ref, t_ref):
        h = acc_ref[...] / (deg_ref[:, 0:1] + EPS)
        t_ref[...] = jnp.tanh(
            jnp.dot(h, w_ref[...], preferred_element_type=jnp.float32)
            + b_ref[...])

    return pl.pallas_call(
        body,
        grid=(R // TR,),
        in_specs=[
            pl.BlockSpec((TR, D), lambda i: (i, 0)),
            pl.BlockSpec((TR, CNT_W), lambda i: (i, 0)),
            pl.BlockSpec((D, D), lambda i: (0, 0)),
            pl.BlockSpec((1, D), lambda i: (0, 0)),
        ],
        out_specs=pl.BlockSpec((TR, D), lambda i: (i, 0)),
        out_shape=jax.ShapeDtypeStruct((R, D), jnp.float32),
    )(acc, deg, w, b)


def _tc_final(acc, deg):
    def body(acc_ref, deg_ref, h_ref):
        h_ref[...] = acc_ref[...] / (deg_ref[:, 0:1] + EPS)

    return pl.pallas_call(
        body,
        grid=(R // TR,),
        in_specs=[
            pl.BlockSpec((TR, D), lambda i: (i, 0)),
            pl.BlockSpec((TR, CNT_W), lambda i: (i, 0)),
        ],
        out_specs=pl.BlockSpec((TR, D), lambda i: (i, 0)),
        out_shape=jax.ShapeDtypeStruct((R, D), jnp.float32),
    )(acc, deg)


def _tc_num(x, w0, b0, w1, b1):
    def body(x_ref, w0_ref, b0_ref, w1_ref, b1_ref, o_ref):
        h = jnp.tanh(jnp.dot(x_ref[...], w0_ref[...],
                             preferred_element_type=jnp.float32) + b0_ref[...])
        o_ref[...] = jnp.tanh(
            jnp.dot(h, w1_ref[...], preferred_element_type=jnp.float32)
            + b1_ref[...])

    return pl.pallas_call(
        body,
        out_shape=jax.ShapeDtypeStruct((8, w1.shape[1]), jnp.float32),
    )(x, w0, b0, w1, b1)


def _prep_edges(ei):
    idx0 = ei[:, :, 0]
    idx1 = ei[:, :, 1]
    pad_s = jnp.full((B, E_PAD - E), TRASH, jnp.int32)
    pad_g = jnp.zeros((B, E_PAD - E), jnp.int32)
    base = (jnp.arange(B, dtype=jnp.int32) * N_PAD)[:, None]
    g0 = (jnp.concatenate([idx1, pad_g], axis=1) + base).reshape(B, NS, K, CH)
    g1 = (jnp.concatenate([idx0, pad_g], axis=1) + base).reshape(B, NS, K, CH)
    s0 = jnp.concatenate([idx0, pad_s], axis=1).reshape(B, NS, K, CH)
    s1 = jnp.concatenate([idx1, pad_s], axis=1).reshape(B, NS, K, CH)
    gat = jnp.concatenate([g0, g1], axis=2)
    sct = jnp.concatenate([s0, s1], axis=2)
    return gat, sct


def kernel(numerical, node_feature, edge_index_dis, edge_index_od,
           W_num0, b_num0, W_num1, b_num1, W_node, b_node,
           W_e1, b_e1, W_e2, b_e2):
    f32 = jnp.float32

    gat_dis, sct_dis = _prep_edges(edge_index_dis)
    gat_od, sct_od = _prep_edges(edge_index_od)
    cnt_idx = jnp.stack([sct_dis, sct_od], axis=0)
    zeros_acc = jnp.zeros((N_PAD, D), f32)
    zeros_cnt = zeros_acc
    ones_rows = jnp.ones((CH, CNT_W), f32)
    nf_pad = jnp.pad(node_feature.astype(f32),
                     ((0, 0), (0, N_PAD - N), (0, 0))).reshape(R, NODE_DIM)
    num_pad = jnp.pad(numerical.astype(f32), ((0, 8 - B), (0, 0)))

    h_num = _tc_num(num_pad, W_num0, b_num0.reshape(1, -1),
                    W_num1, b_num1.reshape(1, -1))[:B]

    deg = _sc_degree(cnt_idx, zeros_cnt, ones_rows)
    deg_dis = deg[0].reshape(R, CNT_W)
    deg_od = deg[1].reshape(R, CNT_W)

    ws = [W_e1[0], W_e2[0], W_e1[1], W_e2[1]]
    bs = [b_e1[0].reshape(1, -1), b_e2[0].reshape(1, -1),
          b_e1[1].reshape(1, -1), b_e2[1].reshape(1, -1)]
    edges = [(gat_dis, sct_dis), (gat_od, sct_od),
             (gat_dis, sct_dis), (gat_od, sct_od)]
    degs = [deg_dis, deg_od, deg_dis, deg_od]

    t = _tc_init(nf_pad, W_node, b_node.reshape(1, -1), ws[0], bs[0])
    h = None
    for k in range(4):
        g, s = edges[k]
        acc = _sc_aggregate(t, g, s, zeros_acc).reshape(R, D)
        if k < 3:
            t = _tc_layer(acc, degs[k], ws[k + 1], bs[k + 1])
        else:
            h = _tc_final(acc, degs[k])

    h_nodes = h.reshape(B, N_PAD, D)[:, :N, :]
    return (h_nodes, h_num)


# speedup vs baseline: 1.1779x; 1.1779x over previous
import functools

import jax
import jax.numpy as jnp
from jax import lax
from jax.experimental import pallas as pl
from jax.experimental.pallas import tpu as pltpu
from jax.experimental.pallas import tpu_sc as plsc

B = 2
N = 10000
E = 160000
D = 128
NODE_DIM = 32
EPS = 1e-6

NC = 2
NS = 16
CH = 128
K = 79
E_PAD = NS * K * CH
N_PAD = 10112
ROWS_PER_SUB = N_PAD // NS
TRASH = N
R = B * N_PAD
CNT_W = 128


def _sc_mesh():
    return plsc.VectorSubcoreMesh(core_axis_name="c", subcore_axis_name="s")


def _sc_aggregate(t_flat, gat_idx, sct_idx, zeros_acc):
    @functools.partial(
        pl.kernel,
        mesh=_sc_mesh(),
        out_type=jax.ShapeDtypeStruct((B, N_PAD, D), jnp.float32),
        scratch_types=[
            pltpu.VMEM((K, CH), jnp.int32),
            pltpu.VMEM((K, CH), jnp.int32),
            pltpu.VMEM((CH, D), jnp.float32),
            pltpu.VMEM_SHARED((N_PAD, D), jnp.float32),
        ],
    )
    def k(t_hbm, g_hbm, s_hbm, z_hbm, out_hbm, gi_v, si_v, rows_v, acc_sh):
        b = lax.axis_index("c")
        s = lax.axis_index("s")
        row0 = s * ROWS_PER_SUB
        pltpu.sync_copy(z_hbm.at[pl.ds(row0, ROWS_PER_SUB)],
                        acc_sh.at[pl.ds(row0, ROWS_PER_SUB)])
        plsc.subcore_barrier()
        for d in range(2):
            pltpu.sync_copy(g_hbm.at[b].at[d].at[s], gi_v)
            pltpu.sync_copy(s_hbm.at[b].at[d].at[s], si_v)

            @pl.loop(0, K)
            def _(j):
                pltpu.sync_copy(t_hbm.at[gi_v.at[j]], rows_v)
                pltpu.sync_copy(rows_v, acc_sh.at[si_v.at[j]], add=True)

        plsc.subcore_barrier()
        pltpu.sync_copy(acc_sh.at[pl.ds(row0, ROWS_PER_SUB)],
                        out_hbm.at[b].at[pl.ds(row0, ROWS_PER_SUB)])

    return k(t_flat, gat_idx, sct_idx, zeros_acc)


def _sc_degree(cnt_idx, zeros_cnt, ones_rows):
    @functools.partial(
        pl.kernel,
        mesh=_sc_mesh(),
        out_type=jax.ShapeDtypeStruct((2, B, N_PAD, CNT_W), jnp.float32),
        scratch_types=[
            pltpu.VMEM((K, CH), jnp.int32),
            pltpu.VMEM((CH, CNT_W), jnp.float32),
            pltpu.VMEM_SHARED((N_PAD, CNT_W), jnp.float32),
        ],
    )
    def k(i_hbm, z_hbm, o_hbm, out_hbm, idx_v, ones_v, cnt_sh):
        b = lax.axis_index("c")
        s = lax.axis_index("s")
        row0 = s * ROWS_PER_SUB
        pltpu.sync_copy(o_hbm, ones_v)
        for e in range(2):
            pltpu.sync_copy(z_hbm.at[pl.ds(row0, ROWS_PER_SUB)],
                            cnt_sh.at[pl.ds(row0, ROWS_PER_SUB)])
            plsc.subcore_barrier()
            for d in range(2):
                pltpu.sync_copy(i_hbm.at[e].at[b].at[d].at[s], idx_v)

                @pl.loop(0, K)
                def _(j):
                    pltpu.sync_copy(ones_v, cnt_sh.at[idx_v.at[j]], add=True)

            plsc.subcore_barrier()
            pltpu.sync_copy(cnt_sh.at[pl.ds(row0, ROWS_PER_SUB)],
                            out_hbm.at[e].at[b].at[pl.ds(row0, ROWS_PER_SUB)])

    return k(cnt_idx, zeros_cnt, ones_rows)


TR = 1264


def _tc_init(nf, w_node, b_node, w0, b0):
    def body(nf_ref, wn_ref, bn_ref, w0_ref, b0_ref, t_ref):
        h = jnp.dot(nf_ref[...], wn_ref[...],
                    preferred_element_type=jnp.float32) + bn_ref[...]
        t_ref[...] = jnp.tanh(
            jnp.dot(h, w0_ref[...], preferred_element_type=jnp.float32)
            + b0_ref[...])

    return pl.pallas_call(
        body,
        grid=(R // TR,),
        in_specs=[
            pl.BlockSpec((TR, NODE_DIM), lambda i: (i, 0)),
            pl.BlockSpec((NODE_DIM, D), lambda i: (0, 0)),
            pl.BlockSpec((1, D), lambda i: (0, 0)),
            pl.BlockSpec((D, D), lambda i: (0, 0)),
            pl.BlockSpec((1, D), lambda i: (0, 0)),
        ],
        out_specs=pl.BlockSpec((TR, D), lambda i: (i, 0)),
        out_shape=jax.ShapeDtypeStruct((R, D), jnp.float32),
    )(nf, w_node, b_node, w0, b0)


def _tc_layer(acc, deg, w, b):
    def body(acc_ref, deg_ref, w_ref, b_ref, t_ref):
        h = acc_ref[...] / (deg_ref[:, 0:1] + EPS)
        t_ref[...] = jnp.tanh(
            jnp.dot(h, w_ref[...], preferred_element_type=jnp.float32)
            + b_ref[...])

    return pl.pallas_call(
        body,
        grid=(R // TR,),
        in_specs=[
            pl.BlockSpec((TR, D), lambda i: (i, 0)),
            pl.BlockSpec((TR, CNT_W), lambda i: (i, 0)),
            pl.BlockSpec((D, D), lambda i: (0, 0)),
            pl.BlockSpec((1, D), lambda i: (0, 0)),
        ],
        out_specs=pl.BlockSpec((TR, D), lambda i: (i, 0)),
        out_shape=jax.ShapeDtypeStruct((R, D), jnp.float32),
    )(acc, deg, w, b)


def _tc_final(acc, deg):
    def body(acc_ref, deg_ref, h_ref):
        h_ref[...] = acc_ref[...] / (deg_ref[:, 0:1] + EPS)

    return pl.pallas_call(
        body,
        grid=(R // TR,),
        in_specs=[
            pl.BlockSpec((TR, D), lambda i: (i, 0)),
            pl.BlockSpec((TR, CNT_W), lambda i: (i, 0)),
        ],
        out_specs=pl.BlockSpec((TR, D), lambda i: (i, 0)),
        out_shape=jax.ShapeDtypeStruct((R, D), jnp.float32),
    )(acc, deg)


def _tc_num(x, w0, b0, w1, b1):
    def body(x_ref, w0_ref, b0_ref, w1_ref, b1_ref, o_ref):
        h = jnp.tanh(jnp.dot(x_ref[...], w0_ref[...],
                             preferred_element_type=jnp.float32) + b0_ref[...])
        o_ref[...] = jnp.tanh(
            jnp.dot(h, w1_ref[...], preferred_element_type=jnp.float32)
            + b1_ref[...])

    return pl.pallas_call(
        body,
        out_shape=jax.ShapeDtypeStruct((8, w1.shape[1]), jnp.float32),
    )(x, w0, b0, w1, b1)


def _prep_edges(ei):
    idx0 = ei[:, :, 0]
    idx1 = ei[:, :, 1]
    pad_s = jnp.full((B, E_PAD - E), TRASH, jnp.int32)
    pad_g = jnp.zeros((B, E_PAD - E), jnp.int32)
    base = (jnp.arange(B, dtype=jnp.int32) * N_PAD)[:, None]
    g0 = jnp.concatenate([idx1, pad_g], axis=1) + base
    g1 = jnp.concatenate([idx0, pad_g], axis=1) + base
    s0 = jnp.concatenate([idx0, pad_s], axis=1)
    s1 = jnp.concatenate([idx1, pad_s], axis=1)
    gat = jnp.stack([g0, g1], axis=1).reshape(B, 2, NS, K, CH)
    sct = jnp.stack([s0, s1], axis=1).reshape(B, 2, NS, K, CH)
    return gat, sct


def kernel(numerical, node_feature, edge_index_dis, edge_index_od,
           W_num0, b_num0, W_num1, b_num1, W_node, b_node,
           W_e1, b_e1, W_e2, b_e2):
    f32 = jnp.float32

    gat_dis, sct_dis = _prep_edges(edge_index_dis)
    gat_od, sct_od = _prep_edges(edge_index_od)
    cnt_idx = jnp.stack([sct_dis, sct_od], axis=0)
    zeros_acc = jnp.zeros((N_PAD, D), f32)
    zeros_cnt = zeros_acc
    ones_rows = jnp.ones((CH, CNT_W), f32)
    nf_pad = jnp.pad(node_feature.astype(f32),
                     ((0, 0), (0, N_PAD - N), (0, 0))).reshape(R, NODE_DIM)
    num_pad = jnp.pad(numerical.astype(f32), ((0, 8 - B), (0, 0)))

    h_num = _tc_num(num_pad, W_num0, b_num0.reshape(1, -1),
                    W_num1, b_num1.reshape(1, -1))[:B]

    deg = _sc_degree(cnt_idx, zeros_cnt, ones_rows)
    deg_dis = deg[0].reshape(R, CNT_W)
    deg_od = deg[1].reshape(R, CNT_W)

    ws = [W_e1[0], W_e2[0], W_e1[1], W_e2[1]]
    bs = [b_e1[0].reshape(1, -1), b_e2[0].reshape(1, -1),
          b_e1[1].reshape(1, -1), b_e2[1].reshape(1, -1)]
    edges = [(gat_dis, sct_dis), (gat_od, sct_od),
             (gat_dis, sct_dis), (gat_od, sct_od)]
    degs = [deg_dis, deg_od, deg_dis, deg_od]

    t = _tc_init(nf_pad, W_node, b_node.reshape(1, -1), ws[0], bs[0])
    h = None
    for k in range(4):
        g, s = edges[k]
        acc = _sc_aggregate(t, g, s, zeros_acc).reshape(R, D)
        if k < 3:
            t = _tc_layer(acc, degs[k], ws[k + 1], bs[k + 1])
        else:
            h = _tc_final(acc, degs[k])

    h_nodes = h.reshape(B, N_PAD, D)[:, :N, :]
    return (h_nodes, h_num)


# speedup vs baseline: 1.2119x; 1.0289x over previous
import functools

import jax
import jax.numpy as jnp
from jax import lax
from jax.experimental import pallas as pl
from jax.experimental.pallas import tpu as pltpu
from jax.experimental.pallas import tpu_sc as plsc

B = 2
N = 10000
E = 160000
D = 128
NODE_DIM = 32
EPS = 1e-6

NC = 2
NS = 16
CH = 128
K = 79
E_PAD = NS * K * CH
N_PAD = 10112
ROWS_PER_SUB = N_PAD // NS
TRASH = N
R = B * N_PAD
CNT_W = 128


def _sc_mesh():
    return plsc.VectorSubcoreMesh(core_axis_name="c", subcore_axis_name="s")


def _sc_aggregate(t_flat, gat_idx, sct_idx, zeros_acc):
    @functools.partial(
        pl.kernel,
        mesh=_sc_mesh(),
        out_type=jax.ShapeDtypeStruct((B, N_PAD, D), jnp.float32),
        scratch_types=[
            pltpu.VMEM((K, CH), jnp.int32),
            pltpu.VMEM((K, CH), jnp.int32),
            pltpu.VMEM((CH, D), jnp.float32),
            pltpu.VMEM_SHARED((N_PAD, D), jnp.float32),
        ],
    )
    def k(t_hbm, g_hbm, s_hbm, z_hbm, out_hbm, gi_v, si_v, rows_v, acc_sh):
        b = lax.axis_index("c")
        s = lax.axis_index("s")
        row0 = s * ROWS_PER_SUB
        pltpu.sync_copy(z_hbm.at[pl.ds(row0, ROWS_PER_SUB)],
                        acc_sh.at[pl.ds(row0, ROWS_PER_SUB)])
        plsc.subcore_barrier()
        for d in range(2):
            pltpu.sync_copy(g_hbm.at[b].at[d].at[s], gi_v)
            pltpu.sync_copy(s_hbm.at[b].at[d].at[s], si_v)

            @pl.loop(0, K)
            def _(j):
                pltpu.sync_copy(t_hbm.at[gi_v.at[j]], rows_v)
                pltpu.sync_copy(rows_v, acc_sh.at[si_v.at[j]], add=True)

        plsc.subcore_barrier()
        pltpu.sync_copy(acc_sh.at[pl.ds(row0, ROWS_PER_SUB)],
                        out_hbm.at[b].at[pl.ds(row0, ROWS_PER_SUB)])

    return k(t_flat, gat_idx, sct_idx, zeros_acc)


def _sc_degree(cnt_idx, zeros_cnt, ones_rows):
    @functools.partial(
        pl.kernel,
        mesh=_sc_mesh(),
        out_type=jax.ShapeDtypeStruct((2, B, N_PAD, CNT_W), jnp.float32),
        scratch_types=[
            pltpu.VMEM((K, CH), jnp.int32),
            pltpu.VMEM((CH, CNT_W), jnp.float32),
            pltpu.VMEM_SHARED((N_PAD, CNT_W), jnp.float32),
        ],
    )
    def k(i_hbm, z_hbm, o_hbm, out_hbm, idx_v, ones_v, cnt_sh):
        b = lax.axis_index("c")
        s = lax.axis_index("s")
        row0 = s * ROWS_PER_SUB
        pltpu.sync_copy(o_hbm, ones_v)
        for e in range(2):
            pltpu.sync_copy(z_hbm.at[pl.ds(row0, ROWS_PER_SUB)],
                            cnt_sh.at[pl.ds(row0, ROWS_PER_SUB)])
            plsc.subcore_barrier()
            for d in range(2):
                pltpu.sync_copy(i_hbm.at[e].at[b].at[d].at[s], idx_v)

                @pl.loop(0, K)
                def _(j):
                    pltpu.sync_copy(ones_v, cnt_sh.at[idx_v.at[j]], add=True)

            plsc.subcore_barrier()
            pltpu.sync_copy(cnt_sh.at[pl.ds(row0, ROWS_PER_SUB)],
                            out_hbm.at[e].at[b].at[pl.ds(row0, ROWS_PER_SUB)])

    return k(cnt_idx, zeros_cnt, ones_rows)


TR = 1264


def _tc_init(nf, w_node, b_node, w0, b0):
    def body(nf_ref, wn_ref, bn_ref, w0_ref, b0_ref, t_ref):
        h = jnp.dot(nf_ref[...], wn_ref[...],
                    preferred_element_type=jnp.float32) + bn_ref[...]
        t_ref[...] = jnp.tanh(
            jnp.dot(h, w0_ref[...], preferred_element_type=jnp.float32)
            + b0_ref[...])

    return pl.pallas_call(
        body,
        grid=(R // TR,),
        in_specs=[
            pl.BlockSpec((TR, NODE_DIM), lambda i: (i, 0)),
            pl.BlockSpec((NODE_DIM, D), lambda i: (0, 0)),
            pl.BlockSpec((1, D), lambda i: (0, 0)),
            pl.BlockSpec((D, D), lambda i: (0, 0)),
            pl.BlockSpec((1, D), lambda i: (0, 0)),
        ],
        out_specs=pl.BlockSpec((TR, D), lambda i: (i, 0)),
        out_shape=jax.ShapeDtypeStruct((R, D), jnp.float32),
    )(nf, w_node, b_node, w0, b0)


def _tc_layer(acc, deg, w, b):
    def body(acc_ref, deg_ref, w_ref, b_ref, t_ref):
        h = acc_ref[...] / (deg_ref[:, 0:1] + EPS)
        t_ref[...] = jnp.tanh(
            jnp.dot(h, w_ref[...], preferred_element_type=jnp.float32)
            + b_ref[...])

    return pl.pallas_call(
        body,
        grid=(R // TR,),
        in_specs=[
            pl.BlockSpec((TR, D), lambda i: (i, 0)),
            pl.BlockSpec((TR, 8), lambda i: (i, 0)),
            pl.BlockSpec((D, D), lambda i: (0, 0)),
            pl.BlockSpec((1, D), lambda i: (0, 0)),
        ],
        out_specs=pl.BlockSpec((TR, D), lambda i: (i, 0)),
        out_shape=jax.ShapeDtypeStruct((R, D), jnp.float32),
    )(acc, deg, w, b)


def _tc_final(acc, deg):
    def body(acc_ref, deg_ref, h_ref):
        h_ref[...] = acc_ref[...] / (deg_ref[:, 0:1] + EPS)

    return pl.pallas_call(
        body,
        grid=(R // TR,),
        in_specs=[
            pl.BlockSpec((TR, D), lambda i: (i, 0)),
            pl.BlockSpec((TR, 8), lambda i: (i, 0)),
        ],
        out_specs=pl.BlockSpec((TR, D), lambda i: (i, 0)),
        out_shape=jax.ShapeDtypeStruct((R, D), jnp.float32),
    )(acc, deg)


def _tc_num(x, w0, b0, w1, b1):
    def body(x_ref, w0_ref, b0_ref, w1_ref, b1_ref, o_ref):
        h = jnp.tanh(jnp.dot(x_ref[...], w0_ref[...],
                             preferred_element_type=jnp.float32) + b0_ref[...])
        o_ref[...] = jnp.tanh(
            jnp.dot(h, w1_ref[...], preferred_element_type=jnp.float32)
            + b1_ref[...])

    return pl.pallas_call(
        body,
        out_shape=jax.ShapeDtypeStruct((8, w1.shape[1]), jnp.float32),
    )(x, w0, b0, w1, b1)


def _prep_edges(ei):
    idx0 = ei[:, :, 0]
    idx1 = ei[:, :, 1]
    pad_s = jnp.broadcast_to(
        TRASH + (jnp.arange(E_PAD - E, dtype=jnp.int32) % (N_PAD - N)),
        (B, E_PAD - E))
    pad_g = jnp.zeros((B, E_PAD - E), jnp.int32)
    base = (jnp.arange(B, dtype=jnp.int32) * N_PAD)[:, None]
    g0 = jnp.concatenate([idx1, pad_g], axis=1) + base
    g1 = jnp.concatenate([idx0, pad_g], axis=1) + base
    s0 = jnp.concatenate([idx0, pad_s], axis=1)
    s1 = jnp.concatenate([idx1, pad_s], axis=1)
    gat = jnp.stack([g0, g1], axis=1).reshape(B, 2, NS, K, CH)
    sct = jnp.stack([s0, s1], axis=1).reshape(B, 2, NS, K, CH)
    return gat, sct


def kernel(numerical, node_feature, edge_index_dis, edge_index_od,
           W_num0, b_num0, W_num1, b_num1, W_node, b_node,
           W_e1, b_e1, W_e2, b_e2):
    f32 = jnp.float32

    gat_dis, sct_dis = _prep_edges(edge_index_dis)
    gat_od, sct_od = _prep_edges(edge_index_od)
    cnt_idx = jnp.stack([sct_dis, sct_od], axis=0)
    zeros_acc = jnp.zeros((N_PAD, D), f32)
    zeros_cnt = zeros_acc
    ones_rows = jnp.ones((CH, CNT_W), f32)
    nf_pad = jnp.pad(node_feature.astype(f32),
                     ((0, 0), (0, N_PAD - N), (0, 0))).reshape(R, NODE_DIM)
    num_pad = jnp.pad(numerical.astype(f32), ((0, 8 - B), (0, 0)))

    h_num = _tc_num(num_pad, W_num0, b_num0.reshape(1, -1),
                    W_num1, b_num1.reshape(1, -1))[:B]

    deg = _sc_degree(cnt_idx, zeros_cnt, ones_rows)
    deg_dis = deg[0].reshape(R, CNT_W)[:, :8]
    deg_od = deg[1].reshape(R, CNT_W)[:, :8]

    ws = [W_e1[0], W_e2[0], W_e1[1], W_e2[1]]
    bs = [b_e1[0].reshape(1, -1), b_e2[0].reshape(1, -1),
          b_e1[1].reshape(1, -1), b_e2[1].reshape(1, -1)]
    edges = [(gat_dis, sct_dis), (gat_od, sct_od),
             (gat_dis, sct_dis), (gat_od, sct_od)]
    degs = [deg_dis, deg_od, deg_dis, deg_od]

    t = _tc_init(nf_pad, W_node, b_node.reshape(1, -1), ws[0], bs[0])
    h = None
    for k in range(4):
        g, s = edges[k]
        acc = _sc_aggregate(t, g, s, zeros_acc).reshape(R, D)
        if k < 3:
            t = _tc_layer(acc, degs[k], ws[k + 1], bs[k + 1])
        else:
            h = _tc_final(acc, degs[k])

    h_nodes = h.reshape(B, N_PAD, D)[:, :N, :]
    return (h_nodes, h_num)
